# baseline (speedup 1.0000x reference)
import functools

import jax
import jax.numpy as jnp
from jax import lax
from jax.experimental import pallas as pl
from jax.experimental.pallas import tpu as pltpu
from jax.experimental.pallas import tpu_sc as plsc

VOCAB = 100000
SEQ_LEN = 200
EMBED_DIM = 128
BATCH = 1024

NUM_CORES = 2
NUM_SUBCORES = 16
NUM_WORKERS = NUM_CORES * NUM_SUBCORES
BPW = BATCH // NUM_WORKERS
IDX_PER_W = BPW * SEQ_LEN
LANES = 16
GROUPS = EMBED_DIM // LANES
SCALE = float(EMBED_DIM) ** 0.5
NBUF = 3
ROWS_PAD = 208
G0, G1 = 104, 96


def _embed_kernel(idx_hbm, table_hbm, pos_hbm, out_hbm, idx_v, pos_v, buf_v,
                  gsem, osem):
    wid = lax.axis_index("s") * NUM_CORES + lax.axis_index("c")
    b0 = wid * BPW

    pltpu.sync_copy(idx_hbm.at[pl.ds(wid * IDX_PER_W, IDX_PER_W)],
                    idx_v.at[pl.ds(0, IDX_PER_W)])
    pltpu.sync_copy(pos_hbm, pos_v.at[pl.ds(0, SEQ_LEN)])

    def gather_descs(b, slot):
        ib = b * SEQ_LEN
        return (
            pltpu.make_async_copy(table_hbm.at[idx_v.at[pl.ds(ib, G0)]],
                                  buf_v.at[slot, pl.ds(0, G0)], gsem),
            pltpu.make_async_copy(table_hbm.at[idx_v.at[pl.ds(ib + G0, G1)]],
                                  buf_v.at[slot, pl.ds(G0, G1)], gsem),
        )

    def out_desc(b, slot):
        return pltpu.make_async_copy(buf_v.at[slot, pl.ds(0, SEQ_LEN)],
                                     out_hbm.at[b0 + b], osem)

    def start_gather(b, slot):
        for d in gather_descs(b, slot):
            d.start()

    def compute(b, slot):
        ib = b * SEQ_LEN

        def grp_body(i, _):
            idxg = idx_v[pl.ds(ib + i * LANES, LANES)]
            mvec = jnp.where(idxg != 0, 1.0, 0.0).astype(jnp.float32)
            for k in range(LANES):
                s = i * LANES + k
                mv = jnp.full((LANES,), mvec[k], jnp.float32)
                for g in range(GROUPS):
                    sl = pl.ds(g * LANES, LANES)
                    buf_v[slot, s, sl] = (
                        buf_v[slot, s, sl] * SCALE + pos_v[s, sl]) * mv
            return 0

        lax.fori_loop(0, ROWS_PAD // LANES, grp_body, 0)

    start_gather(0, 0)
    start_gather(1, 1)

    def step_body(b, _):
        slot = lax.rem(b, NBUF)
        pslot = lax.rem(b + NBUF - 1, NBUF)
        for d in gather_descs(b, slot):
            d.wait()
        compute(b, slot)
        out_desc(b, slot).start()

        @pl.when(b >= 1)
        def _():
            out_desc(b - 1, pslot).wait()

        @pl.when(b + 2 < BPW)
        def _():
            start_gather(b + 2, pslot)

        return 0

    lax.fori_loop(0, BPW, step_body, 0)
    out_desc(BPW - 1, (BPW - 1) % NBUF).wait()


@jax.jit
def _embed(idx_flat, token_table, position_table):
    mesh = plsc.VectorSubcoreMesh(core_axis_name="c", subcore_axis_name="s")
    kern = functools.partial(
        pl.kernel,
        out_type=jax.ShapeDtypeStruct((BATCH, SEQ_LEN, EMBED_DIM),
                                      jnp.float32),
        mesh=mesh,
        scratch_types=[
            pltpu.VMEM((IDX_PER_W + LANES,), jnp.int32),
            pltpu.VMEM((ROWS_PAD, EMBED_DIM), jnp.float32),
            pltpu.VMEM((NBUF, ROWS_PAD, EMBED_DIM), jnp.float32),
            pltpu.SemaphoreType.DMA,
            pltpu.SemaphoreType.DMA,
        ],
    )(_embed_kernel)
    return kern(idx_flat, token_table, position_table)


def kernel(inputs, token_table, position_table):
    idx_flat = inputs.astype(jnp.int32).reshape(-1)
    return _embed(idx_flat, token_table, position_table)


# speedup vs baseline: 1.1523x; 1.1523x over previous
import functools

import jax
import jax.numpy as jnp
from jax import lax
from jax.experimental import pallas as pl
from jax.experimental.pallas import tpu as pltpu
from jax.experimental.pallas import tpu_sc as plsc

VOCAB = 100000
SEQ_LEN = 200
EMBED_DIM = 128
BATCH = 1024

NUM_CORES = 2
NUM_SUBCORES = 16
NUM_WORKERS = NUM_CORES * NUM_SUBCORES
BPW = BATCH // NUM_WORKERS
IDX_PER_W = BPW * SEQ_LEN
LANES = 16
GROUPS = EMBED_DIM // LANES
SCALE = float(EMBED_DIM) ** 0.5
NBUF = 3
ROWS_PAD = 208
G0, G1 = 104, 96


def _embed_kernel(idx_hbm, table_hbm, pos_hbm, out_hbm, idx_v, pos_v, buf_v,
                  gsem, osem):
    wid = lax.axis_index("s") * NUM_CORES + lax.axis_index("c")
    b0 = wid * BPW

    pltpu.sync_copy(idx_hbm.at[pl.ds(wid * IDX_PER_W, IDX_PER_W)],
                    idx_v.at[pl.ds(0, IDX_PER_W)])
    pltpu.sync_copy(pos_hbm, pos_v.at[pl.ds(0, SEQ_LEN)])

    def gather_descs(b, slot):
        ib = b * SEQ_LEN
        r0 = slot * ROWS_PAD
        return (
            pltpu.make_async_copy(table_hbm.at[idx_v.at[pl.ds(ib, G0)]],
                                  buf_v.at[pl.ds(r0, G0)], gsem),
            pltpu.make_async_copy(table_hbm.at[idx_v.at[pl.ds(ib + G0, G1)]],
                                  buf_v.at[pl.ds(r0 + G0, G1)], gsem),
        )

    def out_desc(b, slot):
        return pltpu.make_async_copy(buf_v.at[pl.ds(slot * ROWS_PAD, SEQ_LEN)],
                                     out_hbm.at[b0 + b], osem)

    def start_gather(b, slot):
        for d in gather_descs(b, slot):
            d.start()

    def compute(b, slot):
        ib = b * SEQ_LEN
        r0 = slot * ROWS_PAD

        def grp_body(i, _):
            idxg = idx_v[pl.ds(ib + i * LANES, LANES)]
            mvec = jnp.where(idxg != 0, 1.0, 0.0).astype(jnp.float32)
            for k in range(LANES):
                s = i * LANES + k
                mv = jnp.full((LANES,), mvec[k], jnp.float32)
                for g in range(GROUPS):
                    sl = pl.ds(g * LANES, LANES)
                    buf_v[r0 + s, sl] = (
                        buf_v[r0 + s, sl] * SCALE + pos_v[s, sl]) * mv
            return 0

        lax.fori_loop(0, ROWS_PAD // LANES, grp_body, 0)

    start_gather(0, 0)
    start_gather(1, 1)

    def step_body(b, _):
        slot = lax.rem(b, NBUF)
        pslot = lax.rem(b + NBUF - 1, NBUF)
        for d in gather_descs(b, slot):
            d.wait()
        compute(b, slot)
        out_desc(b, slot).start()

        @pl.when(b >= 1)
        def _():
            out_desc(b - 1, pslot).wait()

        @pl.when(b + 2 < BPW)
        def _():
            start_gather(b + 2, pslot)

        return 0

    lax.fori_loop(0, BPW, step_body, 0)
    out_desc(BPW - 1, (BPW - 1) % NBUF).wait()


@jax.jit
def _embed(idx_flat, token_table, position_table):
    mesh = plsc.VectorSubcoreMesh(core_axis_name="c", subcore_axis_name="s")
    kern = functools.partial(
        pl.kernel,
        out_type=jax.ShapeDtypeStruct((BATCH, SEQ_LEN, EMBED_DIM),
                                      jnp.float32),
        mesh=mesh,
        scratch_types=[
            pltpu.VMEM((IDX_PER_W + LANES,), jnp.int32),
            pltpu.VMEM((ROWS_PAD, EMBED_DIM), jnp.float32),
            pltpu.VMEM((NBUF * ROWS_PAD, EMBED_DIM), jnp.float32),
            pltpu.SemaphoreType.DMA,
            pltpu.SemaphoreType.DMA,
        ],
    )(_embed_kernel)
    return kern(idx_flat, token_table, position_table)


def kernel(inputs, token_table, position_table):
    idx_flat = inputs.astype(jnp.int32).reshape(-1)
    return _embed(idx_flat, token_table, position_table)


# speedup vs baseline: 3.7177x; 3.2264x over previous
import functools

import jax
import jax.numpy as jnp
from jax import lax
from jax.experimental import pallas as pl
from jax.experimental.pallas import tpu as pltpu
from jax.experimental.pallas import tpu_sc as plsc

VOCAB = 100000
SEQ_LEN = 200
EMBED_DIM = 128
BATCH = 1024

NUM_CORES = 2
NUM_SUBCORES = 16
NUM_WORKERS = NUM_CORES * NUM_SUBCORES
BPW = BATCH // NUM_WORKERS
IDX_PER_W = BPW * SEQ_LEN
LANES = 16
GROUPS = EMBED_DIM // LANES
SCALE = float(EMBED_DIM) ** 0.5
NBUF = 3
ROWS_PAD = 208
G0, G1 = 104, 96


def _embed_kernel(idx_hbm, table_hbm, pos_hbm, out_hbm, idx_v, pos_v, buf_v,
                  gsem, osem):
    wid = lax.axis_index("s") * NUM_CORES + lax.axis_index("c")
    b0 = wid * BPW

    pltpu.sync_copy(idx_hbm.at[pl.ds(wid * IDX_PER_W, IDX_PER_W)],
                    idx_v.at[pl.ds(0, IDX_PER_W)])
    pltpu.sync_copy(pos_hbm, pos_v.at[pl.ds(0, SEQ_LEN)])

    def gather_descs(b, slot):
        ib = b * SEQ_LEN
        r0 = slot * ROWS_PAD
        return (
            pltpu.make_async_copy(table_hbm.at[idx_v.at[pl.ds(ib, G0)]],
                                  buf_v.at[pl.ds(r0, G0)], gsem),
            pltpu.make_async_copy(table_hbm.at[idx_v.at[pl.ds(ib + G0, G1)]],
                                  buf_v.at[pl.ds(r0 + G0, G1)], gsem),
        )

    def out_desc(b, slot):
        return pltpu.make_async_copy(buf_v.at[pl.ds(slot * ROWS_PAD, SEQ_LEN)],
                                     out_hbm.at[b0 + b], osem)

    def start_gather(b, slot):
        for d in gather_descs(b, slot):
            d.start()

    def compute(b, slot):
        ib = b * SEQ_LEN
        r0 = slot * ROWS_PAD

        @plsc.parallel_loop(0, ROWS_PAD // LANES)
        def grp_body(i):
            idxg = idx_v[pl.ds(ib + i * LANES, LANES)]
            mvec = jnp.where(idxg != 0, 1.0, 0.0).astype(jnp.float32)
            for k in range(LANES):
                s = i * LANES + k
                mv = jnp.full((LANES,), mvec[k], jnp.float32)
                sls = [pl.ds(g * LANES, LANES) for g in range(GROUPS)]
                vals = [buf_v[r0 + s, sl] for sl in sls]
                res = [(v * SCALE + pos_v[s, sl]) * mv
                       for v, sl in zip(vals, sls)]
                for sl, r in zip(sls, res):
                    buf_v[r0 + s, sl] = r

    start_gather(0, 0)
    start_gather(1, 1)

    def step_body(b, _):
        slot = lax.rem(b, NBUF)
        pslot = lax.rem(b + NBUF - 1, NBUF)
        for d in gather_descs(b, slot):
            d.wait()
        compute(b, slot)
        out_desc(b, slot).start()

        @pl.when(b >= 1)
        def _():
            out_desc(b - 1, pslot).wait()

        @pl.when(b + 2 < BPW)
        def _():
            start_gather(b + 2, pslot)

        return 0

    lax.fori_loop(0, BPW, step_body, 0)
    out_desc(BPW - 1, (BPW - 1) % NBUF).wait()


@jax.jit
def _embed(idx_flat, token_table, position_table):
    mesh = plsc.VectorSubcoreMesh(core_axis_name="c", subcore_axis_name="s")
    kern = functools.partial(
        pl.kernel,
        out_type=jax.ShapeDtypeStruct((BATCH, SEQ_LEN, EMBED_DIM),
                                      jnp.float32),
        mesh=mesh,
        scratch_types=[
            pltpu.VMEM((IDX_PER_W + LANES,), jnp.int32),
            pltpu.VMEM((ROWS_PAD, EMBED_DIM), jnp.float32),
            pltpu.VMEM((NBUF * ROWS_PAD, EMBED_DIM), jnp.float32),
            pltpu.SemaphoreType.DMA,
            pltpu.SemaphoreType.DMA,
        ],
    )(_embed_kernel)
    return kern(idx_flat, token_table, position_table)


def kernel(inputs, token_table, position_table):
    idx_flat = inputs.astype(jnp.int32).reshape(-1)
    return _embed(idx_flat, token_table, position_table)
